# Initial kernel scaffold; baseline (speedup 1.0000x reference)
import jax
import jax.numpy as jnp
from jax.experimental import pallas as pl


def kernel(x, eig_vecs, se, e, edge_index, params):
    raise NotImplementedError("write your pallas kernel here")


# speedup vs baseline: 11.1692x; 11.1692x over previous
import functools

import jax
import jax.numpy as jnp
from jax import lax
from jax.experimental import pallas as pl
from jax.experimental.pallas import tpu as pltpu
from jax.experimental.pallas import tpu_sc as plsc

N = 10000
E = 320000
D = 128
HEADS = 8
DH = 16
ALPHA = 0.5

BN = 1000
NB = N // BN
NW = 32
EPW = E // NW
CH = 40
NCHUNK = EPW // CH
WROW = 144
ZR = 104


def _node_spec(d):
    return pl.BlockSpec((BN, d), lambda i: (i, 0))


def _full_spec(a):
    return pl.BlockSpec(a.shape, lambda i, _nd=a.ndim: (0,) * _nd)


def _ln(x, g, b):
    mu = jnp.mean(x, axis=-1, keepdims=True)
    var = jnp.mean((x - mu) * (x - mu), axis=-1, keepdims=True)
    return (x - mu) * lax.rsqrt(var + 1e-5) * g + b


def _prep_body(x_ref, ev_ref, se_ref, we_ref, be_ref,
               ws0, bs0, ws1, bs1, ws2, bs2, ws3, bs3,
               wp0, bp0, wp1, bp1, wp2, bp2, wp3, bp3,
               out_ref):
    hx = jnp.dot(x_ref[...], we_ref[...],
                 preferred_element_type=jnp.float32) + be_ref[...]
    a = se_ref[...]
    for w, b, act in ((ws0, bs0, 1), (ws1, bs1, 1), (ws2, bs2, 1), (ws3, bs3, 0)):
        a = jnp.dot(a, w[...], preferred_element_type=jnp.float32) + b[...]
        if act:
            a = jnp.maximum(a, 0.0)
    p = ev_ref[...]
    for w, b, act in ((wp0, bp0, 1), (wp1, bp1, 1), (wp2, bp2, 1), (wp3, bp3, 0)):
        p = jnp.dot(p, w[...], preferred_element_type=jnp.float32) + b[...]
        if act:
            p = jnp.maximum(p, 0.0)
    spe = (1.0 - ALPHA) * p + ALPHA * a
    out_ref[...] = jnp.concatenate([hx, spe], axis=1)


def _qkv_body(h_ref, w_ref, b_ref, q_ref, k_ref, v_ref):
    z = jnp.dot(h_ref[...], w_ref[...],
                preferred_element_type=jnp.float32) + b_ref[...]
    q_ref[...] = z[:, 0:D]
    k_ref[...] = z[:, D:2 * D]
    v_ref[...] = z[:, 2 * D:3 * D]


def _post_body(h_ref, p0_ref, p1_ref, wo_ref, bo_ref,
               w1_ref, b1_ref, w2_ref, b2_ref,
               g1_ref, t1_ref, g2_ref, t2_ref, out_ref):
    p = p0_ref[...] + p1_ref[...]
    wv = p[:, 0:D]
    z = p[:, D:D + HEADS]
    colh = lax.broadcasted_iota(jnp.int32, (HEADS, D), 1) // DH
    rowh = lax.broadcasted_iota(jnp.int32, (HEADS, D), 0)
    rmat = (colh == rowh).astype(jnp.float32)
    zden = jnp.dot(z + 1e-6, rmat, preferred_element_type=jnp.float32)
    attn = wv / zden
    h2 = h_ref[...] + jnp.dot(attn, wo_ref[...],
                              preferred_element_type=jnp.float32) + bo_ref[...]
    h2 = _ln(h2, g1_ref[...], t1_ref[...])
    f = jnp.maximum(jnp.dot(h2, w1_ref[...],
                            preferred_element_type=jnp.float32) + b1_ref[...], 0.0)
    f = jnp.dot(f, w2_ref[...], preferred_element_type=jnp.float32) + b2_ref[...]
    out_ref[...] = _ln(h2 + f, g2_ref[...], t2_ref[...])


def _readout_body(h_ref, w1_ref, b1_ref, w2_ref, b2_ref, w3_ref, b3_ref,
                  out_ref):
    hg = jnp.mean(h_ref[...], axis=0, keepdims=True)
    o = jnp.maximum(jnp.dot(hg, w1_ref[...],
                            preferred_element_type=jnp.float32) + b1_ref[...], 0.0)
    o = jnp.maximum(jnp.dot(o, w2_ref[...],
                            preferred_element_type=jnp.float32) + b2_ref[...], 0.0)
    out_ref[...] = jnp.dot(o, w3_ref[...],
                           preferred_element_type=jnp.float32) + b3_ref[...]


@functools.cache
def _build_edge_kernel():
    mesh = plsc.VectorSubcoreMesh(core_axis_name="c", subcore_axis_name="s")

    @functools.partial(
        pl.kernel,
        out_type=jax.ShapeDtypeStruct((2, N, WROW), jnp.float32),
        mesh=mesh,
        compiler_params=pltpu.CompilerParams(
            needs_layout_passes=False, use_tc_tiling_on_sc=False),
        scratch_types=[
            pltpu.VMEM((CH,), jnp.int32),
            pltpu.VMEM((CH,), jnp.int32),
            pltpu.VMEM((CH, D), jnp.float32),
            pltpu.VMEM((CH, D), jnp.float32),
            pltpu.VMEM((CH, D), jnp.float32),
            pltpu.VMEM((CH, WROW), jnp.float32),
            pltpu.VMEM((ZR, WROW), jnp.float32),
            pltpu.VMEM_SHARED((N, WROW), jnp.float32),
            pltpu.SemaphoreType.DMA,
            pltpu.SemaphoreType.DMA,
            pltpu.SemaphoreType.DMA,
        ],
    )
    def _edge_kernel(q_hbm, k_hbm, v_hbm, src_hbm, dst_hbm, out_hbm,
                     src_v, dst_v, krows, qrows, vrows, wrows, zbuf, accum,
                     sk, sq, sv):
        c = lax.axis_index("c")
        s = lax.axis_index("s")
        wid = c * 16 + s
        tile_base = wid * EPW

        zero16 = jnp.zeros((16,), jnp.float32)

        def _zero_zbuf(i, _):
            for j in range(WROW // 16):
                zbuf[i, pl.ds(j * 16, 16)] = zero16
            return 0

        lax.fori_loop(0, ZR, _zero_zbuf, 0)

        def _zero_wrows(i, _):
            for j in range(WROW // 16):
                wrows[i, pl.ds(j * 16, 16)] = zero16
            return 0

        lax.fori_loop(0, CH, _zero_wrows, 0)

        for j in range(6):
            pltpu.sync_copy(zbuf, accum.at[pl.ds(s * 624 + j * ZR, ZR)])

        @pl.when(s == 0)
        def _zero_tail():
            pltpu.sync_copy(zbuf.at[pl.ds(0, 16)], accum.at[pl.ds(9984, 16)])

        plsc.subcore_barrier()

        def _chunk(ci, _):
            base = tile_base + ci * CH
            pltpu.sync_copy(src_hbm.at[pl.ds(base, CH)], src_v)
            pltpu.sync_copy(dst_hbm.at[pl.ds(base, CH)], dst_v)
            cp_k = pltpu.async_copy(k_hbm.at[src_v], krows, sk)
            cp_q = pltpu.async_copy(q_hbm.at[dst_v], qrows, sq)
            cp_v = pltpu.async_copy(v_hbm.at[src_v], vrows, sv)
            cp_k.wait()
            cp_q.wait()
            cp_v.wait()

            for g in range(CH // 8):
                e_idx = lax.broadcasted_iota(jnp.int32, (16,), 0) + g * 16
                for h in range(HEADS):
                    def _dstep(d, acc, _h=h, _e=e_idx):
                        colv = jnp.full((16,), _h * DH, jnp.int32) + d
                        kv = plsc.load_gather(krows, [_e, colv])
                        qv = plsc.load_gather(qrows, [_e, colv])
                        return acc + kv * qv

                    acc = lax.fori_loop(0, DH, _dstep, zero16)
                    sco = jnp.exp(jnp.clip(acc * 0.25, -5.0, 5.0))
                    plsc.store_scatter(
                        wrows, [e_idx, jnp.full((16,), D + h, jnp.int32)], sco)

            def _estep(e, _):
                svec = wrows[e, pl.ds(D, 16)]
                for h in range(HEADS):
                    sval = svec[h]
                    wrows[e, pl.ds(h * DH, DH)] = (
                        vrows[e, pl.ds(h * DH, DH)] * sval)
                return 0

            lax.fori_loop(0, CH, _estep, 0)

            pltpu.sync_copy(wrows, accum.at[dst_v], add=True)
            return 0

        lax.fori_loop(0, NCHUNK, _chunk, 0)

        plsc.subcore_barrier()

        for j in range(6):
            pltpu.sync_copy(accum.at[pl.ds(s * 624 + j * ZR, ZR)],
                            out_hbm.at[c, pl.ds(s * 624 + j * ZR, ZR)])

        @pl.when(s == 0)
        def _copy_tail():
            pltpu.sync_copy(accum.at[pl.ds(9984, 16)],
                            out_hbm.at[c, pl.ds(9984, 16)])

    return _edge_kernel


def _row(b):
    return b.reshape(1, -1)


def kernel(x, eig_vecs, se, e, edge_index, params):
    src = edge_index[0].astype(jnp.int32)
    dst = edge_index[1].astype(jnp.int32)

    mse = params["mlp_se"]
    mpe = params["mlp_pe"]
    prep_ins = [params["emb_h"]["w"], _row(params["emb_h"]["b"])]
    for lyr in mse:
        prep_ins += [lyr["w"], _row(lyr["b"])]
    for lyr in mpe:
        prep_ins += [lyr["w"], _row(lyr["b"])]

    h = pl.pallas_call(
        _prep_body,
        grid=(NB,),
        in_specs=[_node_spec(D), _node_spec(16), _node_spec(16)]
        + [_full_spec(a) for a in prep_ins],
        out_specs=_node_spec(D),
        out_shape=jax.ShapeDtypeStruct((N, D), jnp.float32),
    )(x, eig_vecs, se, *prep_ins)

    for p in params["gt"]:
        wqkv = jnp.concatenate(
            [p["Q"]["w"], p["K"]["w"], p["V"]["w"]], axis=1)
        bqkv = jnp.concatenate(
            [p["Q"]["b"], p["K"]["b"], p["V"]["b"]]).reshape(1, 3 * D)
        q, k, v = pl.pallas_call(
            _qkv_body,
            grid=(NB,),
            in_specs=[_node_spec(D), _full_spec(wqkv), _full_spec(bqkv)],
            out_specs=[_node_spec(D)] * 3,
            out_shape=[jax.ShapeDtypeStruct((N, D), jnp.float32)] * 3,
        )(h, wqkv, bqkv)

        parts = _build_edge_kernel()(q, k, v, src, dst)

        post_ins = [p["O"]["w"], _row(p["O"]["b"]),
                    p["ffn1"]["w"], _row(p["ffn1"]["b"]),
                    p["ffn2"]["w"], _row(p["ffn2"]["b"]),
                    _row(p["ln1_g"]), _row(p["ln1_b"]),
                    _row(p["ln2_g"]), _row(p["ln2_b"])]
        h = pl.pallas_call(
            _post_body,
            grid=(NB,),
            in_specs=[_node_spec(D), _node_spec(WROW), _node_spec(WROW)]
            + [_full_spec(a) for a in post_ins],
            out_specs=_node_spec(D),
            out_shape=jax.ShapeDtypeStruct((N, D), jnp.float32),
        )(h, parts[0], parts[1], *post_ins)

    ro = params["readout"]
    ro_ins = [ro[0]["w"], _row(ro[0]["b"]),
              ro[1]["w"], _row(ro[1]["b"]),
              ro[2]["w"], _row(ro[2]["b"])]
    out = pl.pallas_call(
        _readout_body,
        out_shape=jax.ShapeDtypeStruct((1, 10), jnp.float32),
    )(h, *ro_ins)
    return out
